# baseline (speedup 1.0000x reference)
import functools
import math

import jax
import jax.numpy as jnp
from jax.experimental import pallas as pl
from jax.experimental.pallas import tpu as pltpu


_C1 = 0.7978845608028654
_C3 = 0.7978845608028654 * 0.044715


_W1_SCALE = 64.0
_C1S = _C1 / 64.0
_C3S = _C3 / (64.0 ** 3)


def _mlp_chain(x, w1s_ref, b1_ref, w2s_ref, b2_ref, out_f):
    h = jnp.dot(x.astype(jnp.float8_e4m3fn), w1s_ref[...],
                preferred_element_type=jnp.float32)
    v = (h + b1_ref[...]).astype(jnp.bfloat16)
    t = jnp.tanh(v * (jnp.bfloat16(_C1S) + jnp.bfloat16(_C3S) * (v * v)))
    g = v + v * t
    y = jnp.dot(g, w2s_ref[...],
                preferred_element_type=jnp.float32)
    return x[:, :out_f] + y + b2_ref[...]


def _mlp_kernel(x_ref, w1_ref, b1_ref, w2_ref, b2_ref, o_ref, w1s_ref, w2s_ref,
                *, merge, split):
    @pl.when(pl.program_id(0) == 0)
    def _cast_weights():
        w1s_ref[...] = (w1_ref[...] * _W1_SCALE).astype(jnp.float8_e4m3fn)
        w2s_ref[...] = (w2_ref[...] * (0.5 / _W1_SCALE)).astype(jnp.bfloat16)

    out_f = o_ref.shape[-1]
    sb, rows, in_f = x_ref.shape
    if merge:
        for lo, hi in split:
            x = x_ref[lo:hi].reshape((hi - lo) * rows, in_f)
            o = _mlp_chain(x, w1s_ref, b1_ref, w2s_ref, b2_ref, out_f)
            o_ref[lo:hi] = o.reshape(hi - lo, rows, out_f)
    else:
        for s in range(sb):
            o_ref[s] = _mlp_chain(x_ref[s], w1s_ref, b1_ref, w2s_ref, b2_ref,
                                  out_f)


def kernel(x, w1, b1, w2, b2):
    in_f, hid = w1.shape
    out_f = w2.shape[1]

    if x.ndim == 3:
        x3d = x
    elif x.ndim == 2:
        x3d = x[None]
    else:
        x3d = x.reshape(math.prod(x.shape[:-2]), x.shape[-2], in_f)

    xt = jnp.transpose(x3d, (1, 0, 2))
    lead, rows = xt.shape[0], xt.shape[1]

    sb = next(s for s in (14, 7, 4, 2, 1) if lead % s == 0)
    merge = rows % 8 == 0
    if sb >= 6:
        third = (sb + 2) // 3
        split = ((0, third), (third, 2 * third), (2 * third, sb))
    elif sb > 1:
        half = (sb + 1) // 2
        split = ((0, half), (half, sb))
    else:
        split = ((0, sb),)

    b1r = (b1 * _W1_SCALE).reshape(1, hid)
    b2r = b2.reshape(1, out_f)

    out = pl.pallas_call(
        functools.partial(_mlp_kernel, merge=merge, split=split),
        out_shape=jax.ShapeDtypeStruct((lead, rows, out_f), x.dtype),
        grid=(lead // sb,),
        in_specs=[
            pl.BlockSpec((sb, rows, in_f), lambda i: (i, 0, 0)),
            pl.BlockSpec((in_f, hid), lambda i: (0, 0)),
            pl.BlockSpec((1, hid), lambda i: (0, 0)),
            pl.BlockSpec((hid, out_f), lambda i: (0, 0)),
            pl.BlockSpec((1, out_f), lambda i: (0, 0)),
        ],
        out_specs=pl.BlockSpec((sb, rows, out_f), lambda i: (i, 0, 0)),
        scratch_shapes=[
            pltpu.VMEM((in_f, hid), jnp.float8_e4m3fn),
            pltpu.VMEM((hid, out_f), jnp.bfloat16),
        ],
        compiler_params=pltpu.CompilerParams(
            dimension_semantics=("arbitrary",)),
    )(xt, w1, b1r, w2, b2r)

    out = jnp.transpose(out, (1, 0, 2))
    return out.reshape(x.shape[:-1] + (out_f,))


# speedup vs baseline: 1.0094x; 1.0094x over previous
import functools
import math

import jax
import jax.numpy as jnp
from jax.experimental import pallas as pl
from jax.experimental.pallas import tpu as pltpu


_C1 = 0.7978845608028654
_C3 = 0.7978845608028654 * 0.044715


_W1_SCALE = 64.0


def _mlp_chain(x, w1s_ref, b1_ref, w2s_ref, b2_ref, out_f):
    h = jnp.dot(x.astype(jnp.float8_e4m3fn), w1s_ref[...],
                preferred_element_type=jnp.float32)
    v = (h * (1.0 / _W1_SCALE) + b1_ref[...]).astype(jnp.bfloat16)
    t = jnp.tanh(v * (jnp.bfloat16(_C1) + jnp.bfloat16(_C3) * (v * v)))
    g = v + v * t
    y = jnp.dot(g, w2s_ref[...],
                preferred_element_type=jnp.float32)
    return x[:, :out_f] + y + b2_ref[...]


def _mlp_kernel(x_ref, w1_ref, b1_ref, w2_ref, b2_ref, o_ref, w1s_ref, w2s_ref,
                *, merge, split):
    @pl.when(pl.program_id(0) == 0)
    def _cast_weights():
        w1s_ref[...] = (w1_ref[...] * _W1_SCALE).astype(jnp.float8_e4m3fn)
        w2s_ref[...] = (w2_ref[...] * 0.5).astype(jnp.bfloat16)

    out_f = o_ref.shape[-1]
    sb, rows, in_f = x_ref.shape
    if merge:
        for lo, hi in split:
            x = x_ref[lo:hi].reshape((hi - lo) * rows, in_f)
            o = _mlp_chain(x, w1s_ref, b1_ref, w2s_ref, b2_ref, out_f)
            o_ref[lo:hi] = o.reshape(hi - lo, rows, out_f)
    else:
        for s in range(sb):
            o_ref[s] = _mlp_chain(x_ref[s], w1s_ref, b1_ref, w2s_ref, b2_ref,
                                  out_f)


def kernel(x, w1, b1, w2, b2):
    in_f, hid = w1.shape
    out_f = w2.shape[1]

    if x.ndim == 3:
        x3d = x
    elif x.ndim == 2:
        x3d = x[None]
    else:
        x3d = x.reshape(math.prod(x.shape[:-2]), x.shape[-2], in_f)

    xt = jnp.transpose(x3d, (1, 0, 2))
    lead, rows = xt.shape[0], xt.shape[1]

    sb = next(s for s in (7, 4, 2, 1) if lead % s == 0)
    merge = rows % 8 == 0
    if sb >= 6:
        third = (sb + 2) // 3
        split = ((0, third), (third, 2 * third), (2 * third, sb))
    elif sb > 1:
        half = (sb + 1) // 2
        split = ((0, half), (half, sb))
    else:
        split = ((0, sb),)

    b1r = b1.reshape(1, hid)
    b2r = b2.reshape(1, out_f)

    out = pl.pallas_call(
        functools.partial(_mlp_kernel, merge=merge, split=split),
        out_shape=jax.ShapeDtypeStruct((lead, rows, out_f), x.dtype),
        grid=(lead // sb,),
        in_specs=[
            pl.BlockSpec((sb, rows, in_f), lambda i: (i, 0, 0)),
            pl.BlockSpec((in_f, hid), lambda i: (0, 0)),
            pl.BlockSpec((1, hid), lambda i: (0, 0)),
            pl.BlockSpec((hid, out_f), lambda i: (0, 0)),
            pl.BlockSpec((1, out_f), lambda i: (0, 0)),
        ],
        out_specs=pl.BlockSpec((sb, rows, out_f), lambda i: (i, 0, 0)),
        scratch_shapes=[
            pltpu.VMEM((in_f, hid), jnp.float8_e4m3fn),
            pltpu.VMEM((hid, out_f), jnp.bfloat16),
        ],
        compiler_params=pltpu.CompilerParams(
            dimension_semantics=("arbitrary",)),
    )(xt, w1, b1r, w2, b2r)

    out = jnp.transpose(out, (1, 0, 2))
    return out.reshape(x.shape[:-1] + (out_f,))
